# baseline (speedup 1.0000x reference)
import functools

import jax
import jax.numpy as jnp
from jax import lax
from jax.experimental import pallas as pl
from jax.experimental.pallas import tpu as pltpu
from jax.experimental.pallas import tpu_sc as plsc

B, NCOL, NOUT = 4096, 9, 56
NREF = NCOL - 1
BETA = 10.0
LANES = 16
NC, NS = 2, 16
NW = NC * NS
BPW = B // NW
NG = BPW // LANES
PAIRS = tuple((q, r) for q in range(4) for r in range(1, 4) if q != r)
NPAIR = len(PAIRS)


def _sc_body(idx_hbm, g0_hbm, g1_hbm, sel_hbm, e_hbm, out_hbm,
             idx_v, g0_v, g1_v, sel_v, e_v, stab, out_v, giv, sem):
    wid = lax.axis_index("s") * NC + lax.axis_index("c")
    base = wid * BPW

    lanes = lax.iota(jnp.int32, LANES)
    def fill_gi(k, carry):
        co = jnp.minimum(k * LANES + lanes, NCOL * NOUT - 1)
        giv[pl.ds(k * LANES, LANES)] = co * NW + wid
        return carry
    lax.fori_loop(0, (NCOL * NOUT + LANES - 1) // LANES, fill_gi, 0)
    pltpu.async_copy(idx_hbm.at[giv], idx_v, sem).wait()
    pltpu.sync_copy(g0_hbm.at[:, pl.ds(base, BPW)], g0_v)
    pltpu.sync_copy(g1_hbm.at[:, pl.ds(base, BPW)], g1_v)
    pltpu.sync_copy(sel_hbm.at[:, pl.ds(base, BPW)], sel_v)
    pltpu.sync_copy(e_hbm, e_v)
    zeros_i = jnp.zeros((LANES,), jnp.int32)
    ones_f = jnp.ones((LANES,), jnp.float32)
    pq0 = lax.shift_right_logical(lanes, 2)
    pr0 = lanes & 3
    def16 = ((pq0 == pr0) & (pr0 != 0)).astype(jnp.float32)
    lb16 = lanes * LANES

    def build_tables(g, carry):
        bsl = pl.ds(g * LANES, LANES)
        g00 = g0_v[0, bsl]
        g01 = g0_v[1, bsl]
        g10 = g1_v[0, bsl]
        g11 = g1_v[1, bsl]
        h1 = g00 * g10
        h2 = g00 * g11
        h3 = g01 * g10
        h4 = g01 * g11
        lbase = lb16 + g * (LANES * LANES)
        for k in range(LANES):
            stab[pl.ds(g * (LANES * LANES) + k * LANES, LANES)] = def16
        for p in range(NPAIR):
            eb = p * 8 + 8
            da0 = plsc.load_gather(e_v, [zeros_i + eb])
            db0 = plsc.load_gather(e_v, [zeros_i + (eb + 1)])
            dc0 = plsc.load_gather(e_v, [zeros_i + (eb + 2)])
            dd0 = plsc.load_gather(e_v, [zeros_i + (eb + 3)])
            da1 = plsc.load_gather(e_v, [zeros_i + (eb + 4)])
            db1 = plsc.load_gather(e_v, [zeros_i + (eb + 5)])
            dc1 = plsc.load_gather(e_v, [zeros_i + (eb + 6)])
            dd1 = plsc.load_gather(e_v, [zeros_i + (eb + 7)])
            dz0 = h1 * da0 + h2 * db0 + h3 * dc0 + h4 * dd0
            dz1 = h1 * da1 + h2 * db1 + h3 * dc1 + h4 * dd1
            d2 = dz0 * dz0 + dz1 * dz1
            d2c = jnp.maximum(d2, 1e-30)
            yi = jnp.int32(0x5F3759DF) - lax.shift_right_logical(
                plsc.bitcast(d2c, jnp.int32), 1)
            y = plsc.bitcast(yi, jnp.float32)
            y = y * (1.5 - 0.5 * d2c * y * y)
            y = y * (1.5 - 0.5 * d2c * y * y)
            y = y * (1.5 - 0.5 * d2c * y * y)
            d = d2 * y
            s = jnp.exp(-BETA * d)
            q, r = PAIRS[p]
            plsc.store_scatter(stab, [lbase + (q * 4 + r)], s)
        return carry

    lax.fori_loop(0, NG, build_tables, 0)

    def rank_group(g, carry):
        bsl = pl.ds(g * LANES, LANES)
        selv = [sel_v[c, bsl] > 0.5 for c in range(1, NCOL)]
        lbase = lb16 + g * (LANES * LANES)

        def ostep(o, carry2):
            qv = idx_v[o, bsl]
            qs = lax.shift_left(qv, 2) + lbase
            denom = jnp.zeros((LANES,), jnp.float32)
            prod = ones_f
            for half in range(2):
                num = ones_f
                den = ones_f
                for c in range(NREF - 4 * half, NREF - 4 * (half + 1), -1):
                    rv = idx_v[c * NOUT + o, bsl]
                    sv = plsc.load_gather(stab, [qs | rv])
                    denom = denom + sv
                    num = num * jnp.where(selv[c - 1], sv, 1.0)
                    den = den * jnp.where(selv[c - 1], denom, 1.0)
                prod = prod * (num / den)
            prod = jnp.where(prod > 0.0, prod, 0.0)
            out_v[o, bsl] = prod
            return carry2

        lax.fori_loop(0, NOUT, ostep, 0)
        return carry

    lax.fori_loop(0, NG, rank_group, 0)
    pltpu.sync_copy(out_v, out_hbm.at[:, pl.ds(base, BPW)])


@functools.partial(
    pl.kernel,
    out_type=jax.ShapeDtypeStruct((NOUT, B), jnp.float32),
    mesh=plsc.VectorSubcoreMesh(core_axis_name="c", subcore_axis_name="s"),
    compiler_params=pltpu.CompilerParams(needs_layout_passes=False,
                                         use_tc_tiling_on_sc=False),
    scratch_types=[
        pltpu.VMEM((512, BPW), jnp.int32),
        pltpu.VMEM((2, BPW), jnp.float32),
        pltpu.VMEM((2, BPW), jnp.float32),
        pltpu.VMEM((NCOL, BPW), jnp.float32),
        pltpu.VMEM((NPAIR * 8 + 24,), jnp.float32),
        pltpu.VMEM((BPW * LANES,), jnp.float32),
        pltpu.VMEM((NOUT, BPW), jnp.float32),
        pltpu.VMEM((512,), jnp.int32),
        pltpu.SemaphoreType.DMA,
    ],
)
def _rank_model_sc(idx_hbm, g0_hbm, g1_hbm, sel_hbm, e_hbm, out_hbm,
                   idx_v, g0_v, g1_v, sel_v, e_v, stab, out_v, giv, sem):
    _sc_body(idx_hbm, g0_hbm, g1_hbm, sel_hbm, e_hbm, out_hbm,
             idx_v, g0_v, g1_v, sel_v, e_v, stab, out_v, giv, sem)


def kernel(rank_similarity_stimulus_set, rank_similarity_is_select,
           percept_gate_weights_0, percept_gate_weights_1,
           E0, E1, E2, E3, w_minkowski):
    idx_t = jnp.transpose(rank_similarity_stimulus_set, (1, 2, 0)).reshape(
        NCOL * NOUT * NW, BPW)
    g0_t = jnp.transpose(percept_gate_weights_0, (1, 0))
    g1_t = jnp.transpose(percept_gate_weights_1, (1, 0))
    sel_t = jnp.transpose(rank_similarity_is_select[:, :, 0], (1, 0)).astype(
        jnp.float32)

    ws = jnp.sqrt(w_minkowski)
    rows = []
    for q, r in PAIRS:
        for comp in range(2):
            rows.append(jnp.stack([(E0[q, comp] - E0[r, comp]) * ws[comp],
                                   (E1[q, comp] - E1[r, comp]) * ws[comp],
                                   (E2[q, comp] - E2[r, comp]) * ws[comp],
                                   (E3[q, comp] - E3[r, comp]) * ws[comp]]))
    econst = jnp.concatenate(
        [jnp.zeros((8,), jnp.float32)]
        + [jnp.concatenate([rows[2 * p], rows[2 * p + 1]]) for p in range(NPAIR)]
        + [jnp.zeros((16,), jnp.float32)])
    out = _rank_model_sc(idx_t, g0_t, g1_t, sel_t, econst)
    return jnp.transpose(out, (1, 0))


# speedup vs baseline: 3.2447x; 3.2447x over previous
import functools

import numpy as np

import jax
import jax.numpy as jnp
from jax import lax
from jax.experimental import pallas as pl
from jax.experimental.pallas import tpu as pltpu
from jax.experimental.pallas import tpu_sc as plsc

B, NCOL, NOUT = 4096, 9, 56
NREF = NCOL - 1
BETA = 10.0
LANES = 16
NC, NS = 2, 16
NW = NC * NS
BPW = B // NW
NG = BPW // LANES
PAIRS = tuple((q, r) for q in range(4) for r in range(1, 4) if q != r)
NPAIR = len(PAIRS)


def _sc_body(idx_hbm, g0_hbm, g1_hbm, sel_hbm, e_hbm, out_hbm,
             idx_v, g0_v, g1_v, sel_v, e_v, stab, out_v, giv, sem):
    wid = lax.axis_index("s") * NC + lax.axis_index("c")
    base = wid * BPW

    lanes = lax.iota(jnp.int32, LANES)
    def fill_gi(k, carry):
        co = jnp.minimum(k * LANES + lanes, NCOL * NOUT - 1)
        giv[pl.ds(k * LANES, LANES)] = co * NW + wid
        return carry
    lax.fori_loop(0, (NCOL * NOUT + LANES - 1) // LANES, fill_gi, 0)
    pltpu.async_copy(idx_hbm.at[giv], idx_v, sem).wait()
    pltpu.sync_copy(g0_hbm.at[:, pl.ds(base, BPW)], g0_v)
    pltpu.sync_copy(g1_hbm.at[:, pl.ds(base, BPW)], g1_v)
    pltpu.sync_copy(sel_hbm.at[:, pl.ds(base, BPW)], sel_v)
    pltpu.sync_copy(e_hbm, e_v)
    zeros_i = jnp.zeros((LANES,), jnp.int32)
    ones_f = jnp.ones((LANES,), jnp.float32)
    pq0 = lax.shift_right_logical(lanes, 2)
    pr0 = lanes & 3
    def16 = ((pq0 == pr0) & (pr0 != 0)).astype(jnp.float32)
    lb16 = lanes * LANES

    def build_tables(g, carry):
        bsl = pl.ds(g * LANES, LANES)
        g00 = g0_v[0, bsl]
        g01 = g0_v[1, bsl]
        g10 = g1_v[0, bsl]
        g11 = g1_v[1, bsl]
        h1 = g00 * g10
        h2 = g00 * g11
        h3 = g01 * g10
        h4 = g01 * g11
        lbase = lb16 + g * (LANES * LANES)
        for k in range(LANES):
            stab[pl.ds(g * (LANES * LANES) + k * LANES, LANES)] = def16
        for p in range(NPAIR):
            eb = p * 8 + 8
            da0 = plsc.load_gather(e_v, [zeros_i + eb])
            db0 = plsc.load_gather(e_v, [zeros_i + (eb + 1)])
            dc0 = plsc.load_gather(e_v, [zeros_i + (eb + 2)])
            dd0 = plsc.load_gather(e_v, [zeros_i + (eb + 3)])
            da1 = plsc.load_gather(e_v, [zeros_i + (eb + 4)])
            db1 = plsc.load_gather(e_v, [zeros_i + (eb + 5)])
            dc1 = plsc.load_gather(e_v, [zeros_i + (eb + 6)])
            dd1 = plsc.load_gather(e_v, [zeros_i + (eb + 7)])
            dz0 = h1 * da0 + h2 * db0 + h3 * dc0 + h4 * dd0
            dz1 = h1 * da1 + h2 * db1 + h3 * dc1 + h4 * dd1
            d2 = dz0 * dz0 + dz1 * dz1
            d2c = jnp.maximum(d2, 1e-30)
            yi = jnp.int32(0x5F3759DF) - lax.shift_right_logical(
                plsc.bitcast(d2c, jnp.int32), 1)
            y = plsc.bitcast(yi, jnp.float32)
            y = y * (1.5 - 0.5 * d2c * y * y)
            y = y * (1.5 - 0.5 * d2c * y * y)
            y = y * (1.5 - 0.5 * d2c * y * y)
            d = d2 * y
            s = jnp.exp(-BETA * d)
            q, r = PAIRS[p]
            plsc.store_scatter(stab, [lbase + (q * 4 + r)], s)
        return carry

    lax.fori_loop(0, NG, build_tables, 0)

    def rank_group(g, carry):
        bsl = pl.ds(g * LANES, LANES)
        selv = [sel_v[c, bsl] > 0.5 for c in range(1, NCOL)]
        lbase = lb16 + g * (LANES * LANES)

        def ostep(o, carry2):
            qv = idx_v[o, bsl]
            qs = lax.shift_left(qv, 2) + lbase
            denom = jnp.zeros((LANES,), jnp.float32)
            prod = ones_f
            for half in range(2):
                num = ones_f
                den = ones_f
                for c in range(NREF - 4 * half, NREF - 4 * (half + 1), -1):
                    rv = idx_v[c * NOUT + o, bsl]
                    sv = plsc.load_gather(stab, [qs | rv])
                    denom = denom + sv
                    num = num * jnp.where(selv[c - 1], sv, 1.0)
                    den = den * jnp.where(selv[c - 1], denom, 1.0)
                prod = prod * (num / den)
            prod = jnp.where(prod > 0.0, prod, 0.0)
            out_v[o, bsl] = prod
            return carry2

        lax.fori_loop(0, NOUT, ostep, 0)
        return carry

    lax.fori_loop(0, NG, rank_group, 0)
    pltpu.sync_copy(out_v, out_hbm.at[:, pl.ds(base, BPW)])


@functools.partial(
    pl.kernel,
    out_type=jax.ShapeDtypeStruct((NOUT, B), jnp.float32),
    mesh=plsc.VectorSubcoreMesh(core_axis_name="c", subcore_axis_name="s"),
    compiler_params=pltpu.CompilerParams(needs_layout_passes=False,
                                         use_tc_tiling_on_sc=False),
    scratch_types=[
        pltpu.VMEM((512, BPW), jnp.int32),
        pltpu.VMEM((2, BPW), jnp.float32),
        pltpu.VMEM((2, BPW), jnp.float32),
        pltpu.VMEM((NCOL, BPW), jnp.float32),
        pltpu.VMEM((NPAIR * 8 + 24,), jnp.float32),
        pltpu.VMEM((BPW * LANES,), jnp.float32),
        pltpu.VMEM((NOUT, BPW), jnp.float32),
        pltpu.VMEM((512,), jnp.int32),
        pltpu.SemaphoreType.DMA,
    ],
)
def _rank_model_sc(idx_hbm, g0_hbm, g1_hbm, sel_hbm, e_hbm, out_hbm,
                   idx_v, g0_v, g1_v, sel_v, e_v, stab, out_v, giv, sem):
    _sc_body(idx_hbm, g0_hbm, g1_hbm, sel_hbm, e_hbm, out_hbm,
             idx_v, g0_v, g1_v, sel_v, e_v, stab, out_v, giv, sem)


def kernel(rank_similarity_stimulus_set, rank_similarity_is_select,
           percept_gate_weights_0, percept_gate_weights_1,
           E0, E1, E2, E3, w_minkowski):
    idx_t = jnp.transpose(rank_similarity_stimulus_set, (1, 2, 0)).reshape(
        NCOL * NOUT * NW, BPW)
    g0_t = jnp.transpose(percept_gate_weights_0, (1, 0))
    g1_t = jnp.transpose(percept_gate_weights_1, (1, 0))
    sel_t = jnp.transpose(rank_similarity_is_select[:, :, 0], (1, 0)).astype(
        jnp.float32)

    ws = jnp.sqrt(w_minkowski)
    qs_np = np.array([q for q, _ in PAIRS])
    rs_np = np.array([r for _, r in PAIRS])
    et = jnp.stack([E0, E1, E2, E3])
    dmat = (et[:, qs_np, :] - et[:, rs_np, :]) * ws[None, None, :]
    econst = jnp.concatenate([
        jnp.zeros((8,), jnp.float32),
        jnp.transpose(dmat, (1, 2, 0)).reshape(NPAIR * 8),
        jnp.zeros((16,), jnp.float32)])
    out = _rank_model_sc(idx_t, g0_t, g1_t, sel_t, econst)
    return jnp.transpose(out, (1, 0))


# speedup vs baseline: 3.2471x; 1.0007x over previous
import functools

import numpy as np

import jax
import jax.numpy as jnp
from jax import lax
from jax.experimental import pallas as pl
from jax.experimental.pallas import tpu as pltpu
from jax.experimental.pallas import tpu_sc as plsc

B, NCOL, NOUT = 4096, 9, 56
NREF = NCOL - 1
BETA = 10.0
LANES = 16
NC, NS = 2, 16
NW = NC * NS
BPW = B // NW
NG = BPW // LANES
PAIRS = tuple((q, r) for q in range(4) for r in range(1, 4) if q != r)
NPAIR = len(PAIRS)


def _sc_body(idx_hbm, g0_hbm, g1_hbm, sel_hbm, e_hbm, out_hbm,
             idx_v, g0_v, g1_v, sel_v, e_v, stab, out_v, giv, sem):
    wid = lax.axis_index("s") * NC + lax.axis_index("c")
    base = wid * BPW

    lanes = lax.iota(jnp.int32, LANES)
    def fill_gi(k, carry):
        co = jnp.minimum(k * LANES + lanes, NCOL * NOUT - 1)
        giv[pl.ds(k * LANES, LANES)] = co * NW + wid
        return carry
    lax.fori_loop(0, (NCOL * NOUT + LANES - 1) // LANES, fill_gi, 0)
    pltpu.async_copy(idx_hbm.at[giv], idx_v, sem).wait()
    pltpu.sync_copy(g0_hbm.at[:, pl.ds(base, BPW)], g0_v)
    pltpu.sync_copy(g1_hbm.at[:, pl.ds(base, BPW)], g1_v)
    pltpu.sync_copy(sel_hbm.at[:, pl.ds(base, BPW)], sel_v)
    pltpu.sync_copy(e_hbm, e_v)
    zeros_i = jnp.zeros((LANES,), jnp.int32)
    ones_f = jnp.ones((LANES,), jnp.float32)
    pq0 = lax.shift_right_logical(lanes, 2)
    pr0 = lanes & 3
    def16 = ((pq0 == pr0) & (pr0 != 0)).astype(jnp.float32)
    lb16 = lanes * LANES

    def build_tables(g, carry):
        bsl = pl.ds(g * LANES, LANES)
        g00 = g0_v[0, bsl]
        g01 = g0_v[1, bsl]
        g10 = g1_v[0, bsl]
        g11 = g1_v[1, bsl]
        h1 = g00 * g10
        h2 = g00 * g11
        h3 = g01 * g10
        h4 = g01 * g11
        lbase = lb16 + g * (LANES * LANES)
        for k in range(LANES):
            stab[pl.ds(g * (LANES * LANES) + k * LANES, LANES)] = def16
        for p in range(NPAIR):
            eb = p * 8 + 8
            da0 = plsc.load_gather(e_v, [zeros_i + eb])
            db0 = plsc.load_gather(e_v, [zeros_i + (eb + 1)])
            dc0 = plsc.load_gather(e_v, [zeros_i + (eb + 2)])
            dd0 = plsc.load_gather(e_v, [zeros_i + (eb + 3)])
            da1 = plsc.load_gather(e_v, [zeros_i + (eb + 4)])
            db1 = plsc.load_gather(e_v, [zeros_i + (eb + 5)])
            dc1 = plsc.load_gather(e_v, [zeros_i + (eb + 6)])
            dd1 = plsc.load_gather(e_v, [zeros_i + (eb + 7)])
            dz0 = h1 * da0 + h2 * db0 + h3 * dc0 + h4 * dd0
            dz1 = h1 * da1 + h2 * db1 + h3 * dc1 + h4 * dd1
            d2 = dz0 * dz0 + dz1 * dz1
            d2c = jnp.maximum(d2, 1e-30)
            yi = jnp.int32(0x5F3759DF) - lax.shift_right_logical(
                plsc.bitcast(d2c, jnp.int32), 1)
            y = plsc.bitcast(yi, jnp.float32)
            y = y * (1.5 - 0.5 * d2c * y * y)
            y = y * (1.5 - 0.5 * d2c * y * y)
            y = y * (1.5 - 0.5 * d2c * y * y)
            d = d2 * y
            s = jnp.exp(-BETA * d)
            q, r = PAIRS[p]
            plsc.store_scatter(stab, [lbase + (q * 4 + r)], s)
        return carry

    lax.fori_loop(0, NG, build_tables, 0)

    def rank_group(g, carry):
        bsl = pl.ds(g * LANES, LANES)
        selv = [sel_v[c, bsl] > 0.5 for c in range(1, NCOL)]
        lbase = lb16 + g * (LANES * LANES)

        def one_outcome(o):
            qv = idx_v[o, bsl]
            qs = lax.shift_left(qv, 2) + lbase
            denom = jnp.zeros((LANES,), jnp.float32)
            prod = ones_f
            for half in range(2):
                num = ones_f
                den = ones_f
                for c in range(NREF - 4 * half, NREF - 4 * (half + 1), -1):
                    rv = idx_v[c * NOUT + o, bsl]
                    sv = plsc.load_gather(stab, [qs | rv])
                    denom = denom + sv
                    num = num * jnp.where(selv[c - 1], sv, 1.0)
                    den = den * jnp.where(selv[c - 1], denom, 1.0)
                prod = prod * (num / den)
            prod = jnp.where(prod > 0.0, prod, 0.0)
            out_v[o, bsl] = prod

        def ostep(o2, carry2):
            one_outcome(o2 * 2)
            one_outcome(o2 * 2 + 1)
            return carry2

        lax.fori_loop(0, NOUT // 2, ostep, 0)
        return carry

    lax.fori_loop(0, NG, rank_group, 0)
    pltpu.sync_copy(out_v, out_hbm.at[:, pl.ds(base, BPW)])


@functools.partial(
    pl.kernel,
    out_type=jax.ShapeDtypeStruct((NOUT, B), jnp.float32),
    mesh=plsc.VectorSubcoreMesh(core_axis_name="c", subcore_axis_name="s"),
    compiler_params=pltpu.CompilerParams(needs_layout_passes=False,
                                         use_tc_tiling_on_sc=False),
    scratch_types=[
        pltpu.VMEM((512, BPW), jnp.int32),
        pltpu.VMEM((2, BPW), jnp.float32),
        pltpu.VMEM((2, BPW), jnp.float32),
        pltpu.VMEM((NCOL, BPW), jnp.float32),
        pltpu.VMEM((NPAIR * 8 + 24,), jnp.float32),
        pltpu.VMEM((BPW * LANES,), jnp.float32),
        pltpu.VMEM((NOUT, BPW), jnp.float32),
        pltpu.VMEM((512,), jnp.int32),
        pltpu.SemaphoreType.DMA,
    ],
)
def _rank_model_sc(idx_hbm, g0_hbm, g1_hbm, sel_hbm, e_hbm, out_hbm,
                   idx_v, g0_v, g1_v, sel_v, e_v, stab, out_v, giv, sem):
    _sc_body(idx_hbm, g0_hbm, g1_hbm, sel_hbm, e_hbm, out_hbm,
             idx_v, g0_v, g1_v, sel_v, e_v, stab, out_v, giv, sem)


def kernel(rank_similarity_stimulus_set, rank_similarity_is_select,
           percept_gate_weights_0, percept_gate_weights_1,
           E0, E1, E2, E3, w_minkowski):
    idx_t = jnp.transpose(rank_similarity_stimulus_set, (1, 2, 0)).reshape(
        NCOL * NOUT * NW, BPW)
    g0_t = jnp.transpose(percept_gate_weights_0, (1, 0))
    g1_t = jnp.transpose(percept_gate_weights_1, (1, 0))
    sel_t = jnp.transpose(rank_similarity_is_select[:, :, 0], (1, 0)).astype(
        jnp.float32)

    ws = jnp.sqrt(w_minkowski)
    qs_np = np.array([q for q, _ in PAIRS])
    rs_np = np.array([r for _, r in PAIRS])
    et = jnp.stack([E0, E1, E2, E3])
    dmat = (et[:, qs_np, :] - et[:, rs_np, :]) * ws[None, None, :]
    econst = jnp.concatenate([
        jnp.zeros((8,), jnp.float32),
        jnp.transpose(dmat, (1, 2, 0)).reshape(NPAIR * 8),
        jnp.zeros((16,), jnp.float32)])
    out = _rank_model_sc(idx_t, g0_t, g1_t, sel_t, econst)
    return jnp.transpose(out, (1, 0))
